# baseline (speedup 1.0000x reference)
import functools

import jax
import jax.numpy as jnp
from jax import lax
from jax.experimental import pallas as pl
from jax.experimental.pallas import tpu as pltpu
from jax.experimental.pallas import tpu_sc as plsc

B = 4096
N_CAT = 26
N_NUM = 16
N_TOK = N_CAT + N_NUM
D = 32
VOCAB = 100000

NC = 2
NS = 16
NW = NC * NS
L = 16
G8 = 8
NGRP = N_CAT * D // G8
SEG = 3968
NFULL = VOCAB // SEG
TAIL = VOCAB - NFULL * SEG
BPAD = B + L
NUM_SEG = 256


def _sc_body(tab_hbm, idx_hbm, xnum_hbm, w_hbm, bias_hbm, out_hbm,
             segbuf_v, tailbuf_v, outrows_v, idx_v, vlist_v, blist_v,
             numblk_v, xn_v, wb_v,
             sem_a, sem_b, sem_t, sem_idx, sem_out):
    cid = lax.axis_index("c")
    sid = lax.axis_index("s")
    wid = sid * NC + cid
    g_lo = wid * NGRP // NW
    g_hi = (wid + 1) * NGRP // NW

    pltpu.sync_copy(w_hbm, wb_v.at[0])
    pltpu.sync_copy(bias_hbm, wb_v.at[1])
    n = wid // 2
    dh = wid % 2
    for lseg in range(B // NUM_SEG):
        pltpu.sync_copy(xnum_hbm.at[n, pl.ds(lseg * NUM_SEG, NUM_SEG)], xn_v)
        for dj in range(D // 2):
            wrow = wb_v[0, n, pl.ds(dh * L, L)]
            brow = wb_v[1, n, pl.ds(dh * L, L)]
            ws = wrow[dj]
            bs = brow[dj]
            for c in range(NUM_SEG // L):
                x = xn_v[pl.ds(c * L, L)]
                numblk_v[dj, pl.ds(c * L, L)] = x * ws + bs
        pltpu.async_copy(
            numblk_v,
            out_hbm.at[N_CAT + n, pl.ds(dh * (D // 2), D // 2),
                       pl.ds(lseg * NUM_SEG, NUM_SEG)],
            sem_out).wait()

    iota16 = lax.iota(jnp.int32, L)

    def seg_wait(g, p, sem):
        pltpu.make_async_copy(
            tab_hbm.at[g, :, pl.ds(0, SEG)], segbuf_v.at[p], sem).wait()

    def seg_fire(g, s, p, sem):
        return pltpu.async_copy(
            tab_hbm.at[g, :, pl.ds(s * SEG, SEG)], segbuf_v.at[p], sem)

    def build_list(base, seg_len):
        def build(k, cnt):
            iv = idx_v[pl.ds(k * L, L)]
            vrel = iv - base
            m = jnp.logical_and(iv >= base, iv < base + seg_len)
            plsc.store_compressed(vlist_v.at[pl.ds(cnt, L)], vrel, mask=m)
            bpos = iota16 + (k * L)
            plsc.store_compressed(blist_v.at[pl.ds(cnt, L)], bpos, mask=m)
            pc = plsc.all_reduce_population_count(m)
            return cnt + pc[0]
        cnt = lax.fori_loop(0, B // L, build, jnp.int32(0))
        vlist_v[pl.ds(cnt, L)] = jnp.zeros((L,), jnp.int32)
        blist_v[pl.ds(cnt, L)] = jnp.full((L,), B, jnp.int32)
        return (cnt + L - 1) // L

    def gather_seg(nch, p):
        pvec = jnp.full((L,), p, jnp.int32)

        def gather(i, _):
            vrel = vlist_v[pl.ds(i * L, L)]
            bpos = blist_v[pl.ds(i * L, L)]
            for d in range(G8):
                dvec = jnp.full((L,), d, jnp.int32)
                vals = plsc.load_gather(segbuf_v, [pvec, dvec, vrel])
                plsc.store_scatter(outrows_v, [dvec, bpos], vals)
            return _
        lax.fori_loop(0, nch, gather, None)

    def gather_tail(nch):
        def gather(i, _):
            vrel = vlist_v[pl.ds(i * L, L)]
            bpos = blist_v[pl.ds(i * L, L)]
            for d in range(G8):
                dvec = jnp.full((L,), d, jnp.int32)
                vals = plsc.load_gather(tailbuf_v, [dvec, vrel])
                plsc.store_scatter(outrows_v, [dvec, bpos], vals)
            return _
        lax.fori_loop(0, nch, gather, None)

    def group_body(g, _):
        f = g // (D // G8)
        dsub = (g % (D // G8)) * G8
        pltpu.sync_copy(idx_hbm.at[f], idx_v)

        seg_fire(g, 0, 0, sem_a)
        seg_fire(g, 1, 1, sem_b)
        pltpu.async_copy(
            tab_hbm.at[g, :, pl.ds(NFULL * SEG, TAIL)], tailbuf_v, sem_t)

        def pair_body(m, _):
            s0 = 2 * m
            nch = build_list(s0 * SEG, SEG)
            seg_wait(g, 0, sem_a)
            gather_seg(nch, 0)
            seg_fire(g, s0 + 2, 0, sem_a)

            s1 = s0 + 1
            nch = build_list(s1 * SEG, SEG)
            seg_wait(g, 1, sem_b)
            gather_seg(nch, 1)
            sf = jnp.minimum(s1 + 2, NFULL - 1)
            seg_fire(g, sf, 1, sem_b)
            return _
        lax.fori_loop(0, (NFULL - 1) // 2, pair_body, None)

        nch = build_list((NFULL - 1) * SEG, SEG)
        seg_wait(g, 0, sem_a)
        gather_seg(nch, 0)
        seg_wait(g, 1, sem_b)

        nch = build_list(NFULL * SEG, TAIL)
        pltpu.make_async_copy(
            tab_hbm.at[g, :, pl.ds(NFULL * SEG, TAIL)], tailbuf_v,
            sem_t).wait()
        gather_tail(nch)

        ocps = []
        for dd in range(G8):
            ocps.append(pltpu.async_copy(
                outrows_v.at[dd, pl.ds(0, B)],
                out_hbm.at[f, dsub + dd], sem_out))
        for ocp in ocps:
            ocp.wait()
        return _

    lax.fori_loop(g_lo, g_hi, group_body, None)


@jax.jit
def kernel(x_cat, x_num, cat_tables, num_weight, num_bias):
    tab3 = cat_tables.transpose(0, 2, 1).reshape(NGRP, G8, VOCAB)
    idx_t = x_cat.T
    xnum_t = x_num.T

    mesh = plsc.VectorSubcoreMesh(core_axis_name="c", subcore_axis_name="s")
    run = pl.kernel(
        _sc_body,
        out_type=jax.ShapeDtypeStruct((N_TOK, D, B), jnp.float32),
        mesh=mesh,
        compiler_params=pltpu.CompilerParams(needs_layout_passes=False),
        scratch_types=[
            pltpu.VMEM((2, G8, SEG), jnp.float32),
            pltpu.VMEM((G8, TAIL), jnp.float32),
            pltpu.VMEM((G8, BPAD), jnp.float32),
            pltpu.VMEM((B,), jnp.int32),
            pltpu.VMEM((BPAD,), jnp.int32),
            pltpu.VMEM((BPAD,), jnp.int32),
            pltpu.VMEM((D // 2, NUM_SEG), jnp.float32),
            pltpu.VMEM((NUM_SEG,), jnp.float32),
            pltpu.VMEM((2, N_NUM, D), jnp.float32),
            pltpu.SemaphoreType.DMA,
            pltpu.SemaphoreType.DMA,
            pltpu.SemaphoreType.DMA,
            pltpu.SemaphoreType.DMA,
            pltpu.SemaphoreType.DMA,
        ],
    )
    out_t = run(tab3, idx_t, xnum_t, num_weight, num_bias)
    return out_t.transpose(2, 0, 1)
